# baseline (speedup 1.0000x reference)
import functools

import jax
import jax.numpy as jnp
from jax import lax
from jax.experimental import pallas as pl
from jax.experimental.pallas import tpu as pltpu
from jax.experimental.pallas import tpu_sc as plsc

NC = 2
NS = 16
L = 16
NW = NC * NS

B = 16384
A = 100
O = 32
AP = 112
CH = 16
NCHUNK = B // CH // NW
IST = 4


def _body(fi_hbm, fvt_hbm, w_hbm, bias_hbm, out_hbm,
          idx_v, val_v, rows_v, out_v, bias_v, sem0, sem1):
    cid = lax.axis_index("c")
    sid = lax.axis_index("s")
    wid = sid * NC + cid

    iota = lax.iota(jnp.int32, L)
    zeros = jnp.zeros((L,), jnp.float32)
    sems = (sem0, sem1)

    for bb in range(2):
        for j in range(CH):
            for i in range(A, AP):
                for oo in range(O // L):
                    rows_v[bb, j, i, pl.ds(oo * L, L)] = zeros

    pltpu.sync_copy(bias_hbm, bias_v)

    GATHER_BYTES = CH * A * O * 4

    def fetch(k, bb):
        chunk_id = wid * NCHUNK + k
        base = chunk_id * CH
        pltpu.sync_copy(fi_hbm.at[pl.ds(base, CH)], idx_v.at[bb])
        pltpu.sync_copy(fvt_hbm.at[chunk_id], val_v.at[bb])
        for j in range(CH):
            pltpu.async_copy(w_hbm.at[idx_v.at[bb, j]],
                             rows_v.at[bb, j, pl.ds(0, A)], sems[bb])

    def drain(bb):
        for j in range(CH):
            pltpu.make_async_copy(w_hbm.at[pl.ds(0, A)],
                                  rows_v.at[bb, j, pl.ds(0, A)],
                                  sems[bb]).wait()

    def compute(k, bb):
        rows_b = rows_v.at[bb]
        for og in range(O // L):
            def ibody(g, accs):
                new = list(accs)
                for di in range(IST):
                    i = g * IST + di
                    v = val_v[bb, i, :]
                    ii = jnp.broadcast_to(i, (L,))
                    for oo in range(L):
                        o = og * L + oo
                        x = plsc.load_gather(
                            rows_b,
                            [iota, ii, jnp.full((L,), o, jnp.int32)])
                        new[oo] = new[oo] + x * v
                return tuple(new)

            accs0 = tuple(bias_v[og * L + oo, :] for oo in range(L))
            accs = lax.fori_loop(0, AP // IST, ibody, accs0)
            for oo in range(L):
                plsc.store_scatter(
                    out_v, [iota, jnp.full((L,), og * L + oo, jnp.int32)],
                    accs[oo])
        base = (wid * NCHUNK + k) * CH
        pltpu.sync_copy(out_v, out_hbm.at[pl.ds(base, CH)])

    fetch(0, 0)

    @pl.loop(0, NCHUNK, step=2)
    def _pair(k):
        fetch(k + 1, 1)
        drain(0)
        compute(k, 0)

        @pl.when(k + 2 < NCHUNK)
        def _():
            fetch(k + 2, 0)

        drain(1)
        compute(k + 1, 1)


@jax.jit
def _run(fi, fvt, w, bias_b):
    mesh = plsc.VectorSubcoreMesh(core_axis_name="c", subcore_axis_name="s")
    f = pl.kernel(
        _body,
        out_type=jax.ShapeDtypeStruct((B, O), jnp.float32),
        mesh=mesh,
        compiler_params=pltpu.CompilerParams(
            needs_layout_passes=False,
            use_tc_tiling_on_sc=False,
        ),
        scratch_types=[
            pltpu.VMEM((2, CH, A), jnp.int32),
            pltpu.VMEM((2, AP, CH), jnp.float32),
            pltpu.VMEM((2, CH, AP, O), jnp.float32),
            pltpu.VMEM((CH, O), jnp.float32),
            pltpu.VMEM((O, L), jnp.float32),
            pltpu.SemaphoreType.DMA,
            pltpu.SemaphoreType.DMA,
        ],
    )
    return f(fi, fvt, w, bias_b)


def kernel(feature_indices, feature_values, weight, bias):
    fv = jnp.pad(feature_values, ((0, 0), (0, AP - A)))
    fvt = fv.reshape(B // CH, CH, AP).transpose(0, 2, 1)
    bias_b = jnp.broadcast_to(bias[:, None], (O, L))
    return _run(feature_indices, fvt, weight, bias_b)


# speedup vs baseline: 2.1927x; 2.1927x over previous
import functools

import jax
import jax.numpy as jnp
from jax import lax
from jax.experimental import pallas as pl
from jax.experimental.pallas import tpu as pltpu
from jax.experimental.pallas import tpu_sc as plsc

NC = 2
NS = 16
L = 16
NW = NC * NS

B = 16384
A = 100
O = 32
AP = 112
CH = 16
NCHUNK = B // CH // NW
NPART = 4


def _body(fi_hbm, fvp_hbm, w_hbm, bias_hbm, out_hbm,
          idx_v, val_v, rows_v, out_v, bias_v, sem0, sem1):
    cid = lax.axis_index("c")
    sid = lax.axis_index("s")
    wid = sid * NC + cid

    zeros = jnp.zeros((L,), jnp.float32)
    sems = (sem0, sem1)

    for bb in range(2):
        for j in range(CH):
            for i in range(A, AP):
                for oo in range(O // L):
                    rows_v[bb, j, i, pl.ds(oo * L, L)] = zeros

    pltpu.sync_copy(bias_hbm, bias_v)

    def fetch(k, bb):
        chunk_id = wid * NCHUNK + k
        base = chunk_id * CH
        pltpu.sync_copy(fi_hbm.at[pl.ds(base, CH)], idx_v.at[bb])
        pltpu.sync_copy(fvp_hbm.at[chunk_id], val_v.at[bb])
        for j in range(CH):
            pltpu.async_copy(w_hbm.at[idx_v.at[bb, j]],
                             rows_v.at[bb, j, pl.ds(0, A)], sems[bb])

    def drain(bb):
        for j in range(CH):
            pltpu.make_async_copy(w_hbm.at[pl.ds(0, A)],
                                  rows_v.at[bb, j, pl.ds(0, A)],
                                  sems[bb]).wait()

    def compute(k, bb):
        bias0 = bias_v[pl.ds(0, L)]
        bias1 = bias_v[pl.ds(L, L)]

        @pl.loop(0, CH)
        def _row(j):
            p0 = [bias0] + [zeros] * (NPART - 1)
            p1 = [bias1] + [zeros] * (NPART - 1)
            for g in range(AP // L):
                v16 = val_v[bb, j, pl.ds(g * L, L)]
                for di in range(L):
                    i = g * L + di
                    sp = jnp.take_along_axis(
                        v16, jnp.full((L,), di, jnp.int32), axis=0)
                    r0 = rows_v[bb, j, i, pl.ds(0, L)]
                    r1 = rows_v[bb, j, i, pl.ds(L, L)]
                    p0[di % NPART] = p0[di % NPART] + sp * r0
                    p1[di % NPART] = p1[di % NPART] + sp * r1
            out_v[j, pl.ds(0, L)] = (p0[0] + p0[1]) + (p0[2] + p0[3])
            out_v[j, pl.ds(L, L)] = (p1[0] + p1[1]) + (p1[2] + p1[3])

        base = (wid * NCHUNK + k) * CH
        pltpu.sync_copy(out_v, out_hbm.at[pl.ds(base, CH)])

    fetch(0, 0)

    @pl.loop(0, NCHUNK, step=2)
    def _pair(k):
        fetch(k + 1, 1)
        drain(0)
        compute(k, 0)

        @pl.when(k + 2 < NCHUNK)
        def _():
            fetch(k + 2, 0)

        drain(1)
        compute(k + 1, 1)


@jax.jit
def _run(fi, fvp, w, bias):
    mesh = plsc.VectorSubcoreMesh(core_axis_name="c", subcore_axis_name="s")
    f = pl.kernel(
        _body,
        out_type=jax.ShapeDtypeStruct((B, O), jnp.float32),
        mesh=mesh,
        compiler_params=pltpu.CompilerParams(
            needs_layout_passes=False,
            use_tc_tiling_on_sc=False,
        ),
        scratch_types=[
            pltpu.VMEM((2, CH, A), jnp.int32),
            pltpu.VMEM((2, CH, AP), jnp.float32),
            pltpu.VMEM((2, CH, AP, O), jnp.float32),
            pltpu.VMEM((CH, O), jnp.float32),
            pltpu.VMEM((O,), jnp.float32),
            pltpu.SemaphoreType.DMA,
            pltpu.SemaphoreType.DMA,
        ],
    )
    return f(fi, fvp, w, bias)


def kernel(feature_indices, feature_values, weight, bias):
    fvp = jnp.pad(feature_values, ((0, 0), (0, AP - A)))
    fvp = fvp.reshape(B // CH, CH, AP)
    return _run(feature_indices, fvp, weight, bias)


# speedup vs baseline: 2.2287x; 1.0164x over previous
import functools

import jax
import jax.numpy as jnp
from jax import lax
from jax.experimental import pallas as pl
from jax.experimental.pallas import tpu as pltpu
from jax.experimental.pallas import tpu_sc as plsc

NC = 2
NS = 16
L = 16
NW = NC * NS

B = 16384
A = 100
O = 32
CH = 16
NCHUNK = B // CH // NW
NFULL = A // L
NTAIL = A - NFULL * L
NPART = 4


def _body(fi_hbm, fv_hbm, w_hbm, bias_hbm, out_hbm,
          idx_v, val_v, rows_v, out_v, bias_v, sem0, sem1):
    cid = lax.axis_index("c")
    sid = lax.axis_index("s")
    wid = sid * NC + cid

    zeros = jnp.zeros((L,), jnp.float32)
    sems = (sem0, sem1)

    pltpu.sync_copy(bias_hbm, bias_v)

    def fetch(k, bb):
        chunk_id = wid * NCHUNK + k
        base = chunk_id * CH
        pltpu.sync_copy(fi_hbm.at[pl.ds(base, CH)], idx_v.at[bb])
        pltpu.sync_copy(fv_hbm.at[pl.ds(base * A, CH * A)], val_v.at[bb])
        for j in range(CH):
            pltpu.async_copy(w_hbm.at[idx_v.at[bb, j]],
                             rows_v.at[bb, j], sems[bb])

    def drain(bb):
        for j in range(CH):
            pltpu.make_async_copy(w_hbm.at[pl.ds(0, A)],
                                  rows_v.at[bb, j], sems[bb]).wait()

    def compute(k, bb):
        bias0 = bias_v[pl.ds(0, L)]
        bias1 = bias_v[pl.ds(L, L)]

        @pl.loop(0, CH)
        def _row(j):
            vbase = j * A
            p0 = [bias0] + [zeros] * (NPART - 1)
            p1 = [bias1] + [zeros] * (NPART - 1)

            def fma(i, sp):
                r0 = rows_v[bb, j, i, pl.ds(0, L)]
                r1 = rows_v[bb, j, i, pl.ds(L, L)]
                p0[i % NPART] = p0[i % NPART] + sp * r0
                p1[i % NPART] = p1[i % NPART] + sp * r1

            for g in range(NFULL):
                v16 = val_v[bb, pl.ds(vbase + g * L, L)]
                for di in range(L):
                    sp = jnp.take_along_axis(
                        v16, jnp.full((L,), di, jnp.int32), axis=0)
                    fma(g * L + di, sp)
            vt = val_v[bb, pl.ds(vbase + A - L, L)]
            for di in range(L - NTAIL, L):
                sp = jnp.take_along_axis(
                    vt, jnp.full((L,), di, jnp.int32), axis=0)
                fma(A - L + di, sp)

            out_v[j, pl.ds(0, L)] = (p0[0] + p0[1]) + (p0[2] + p0[3])
            out_v[j, pl.ds(L, L)] = (p1[0] + p1[1]) + (p1[2] + p1[3])

        base = (wid * NCHUNK + k) * CH
        pltpu.sync_copy(out_v, out_hbm.at[pl.ds(base, CH)])

    fetch(0, 0)

    @pl.loop(0, NCHUNK, step=2)
    def _pair(k):
        fetch(k + 1, 1)
        drain(0)
        compute(k, 0)

        @pl.when(k + 2 < NCHUNK)
        def _():
            fetch(k + 2, 0)

        drain(1)
        compute(k + 1, 1)


@jax.jit
def _run(fi, fv, w, bias):
    mesh = plsc.VectorSubcoreMesh(core_axis_name="c", subcore_axis_name="s")
    f = pl.kernel(
        _body,
        out_type=jax.ShapeDtypeStruct((B, O), jnp.float32),
        mesh=mesh,
        compiler_params=pltpu.CompilerParams(
            needs_layout_passes=False,
            use_tc_tiling_on_sc=False,
        ),
        scratch_types=[
            pltpu.VMEM((2, CH, A), jnp.int32),
            pltpu.VMEM((2, CH * A), jnp.float32),
            pltpu.VMEM((2, CH, A, O), jnp.float32),
            pltpu.VMEM((CH, O), jnp.float32),
            pltpu.VMEM((O,), jnp.float32),
            pltpu.SemaphoreType.DMA,
            pltpu.SemaphoreType.DMA,
        ],
    )
    return f(fi, fv, w, bias)


def kernel(feature_indices, feature_values, weight, bias):
    fv = feature_values.reshape(B * A)
    return _run(feature_indices, fv, weight, bias)
